# Initial kernel scaffold; baseline (speedup 1.0000x reference)
import jax
import jax.numpy as jnp
from jax.experimental import pallas as pl

_TOP_M = 1024
_KAPPA = 0.05
_XI_H = 0.005
_RHO_F = 0.2
_C_V = 2.0
_LEAK = 0.01
_SIGMA = 2.0
_RADIUS = 4
_THRESH = 0.5


def _gk(sigma, radius):
    x = jnp.arange(-radius, radius + 1, dtype=jnp.float32)
    k = jnp.exp(-0.5 * (x / sigma) ** 2)
    return k / jnp.sum(k)


def _blur_axis(t, kern, axis):
    r = (kern.shape[0] - 1) // 2
    pad = [(0, 0)] * t.ndim
    pad[axis] = (r, r)
    tp = jnp.pad(t, pad)
    L = t.shape[axis]
    out = jnp.zeros_like(t)
    for j in range(kern.shape[0]):
        sl = [slice(None)] * t.ndim
        sl[axis] = slice(j, j + L)
        out = out + kern[j] * tp[tuple(sl)]
    return out


def _bf16_dot(a, b):
    return jax.lax.dot(a.astype(jnp.bfloat16), b.astype(jnp.bfloat16),
                       preferred_element_type=jnp.float32)


def kernel(stm_K, stm_V, stm_e, stm_h, stm_active, ltm_K, ltm_V, ltm_e, ltm_h,
           stm_terrain, ltm_terrain, fatigue, W, b):
    masked_h = jnp.where(stm_active, stm_h, -jnp.inf)
    top_h, top_idx = jax.lax.top_k(masked_h, _TOP_M)
    K_sel = stm_K[top_idx]
    V_sel = stm_V[top_idx]
    e_sel = stm_e[top_idx]
    h_sel = stm_h[top_idx]
    K_proj = _bf16_dot(K_sel, W) + b
    omega = _KAPPA * h_sel
    qn = K_proj / (jnp.linalg.norm(K_proj, axis=-1, keepdims=True) + 1e-6)
    kn = ltm_K / (jnp.linalg.norm(ltm_K, axis=-1, keepdims=True) + 1e-6)
    sim = _bf16_dot(qn, kn.T)
    best_idx = jnp.argmax(sim, axis=-1)
    best_sim = jnp.take_along_axis(sim, best_idx[:, None], axis=-1)[:, 0]
    new_count = jnp.sum(best_sim <= _THRESH)
    alpha = omega[:, None]
    ltm_V_new = ltm_V.at[best_idx].add(alpha * V_sel)
    ltm_K_new = ltm_K.at[best_idx].add(alpha * K_proj)
    ltm_e_new = ltm_e.at[best_idx].add(alpha * e_sel)
    ltm_h_new = ltm_h.at[best_idx].add(omega)
    blurred = _blur_axis(_blur_axis(_blur_axis(stm_terrain, _gk(_SIGMA, _RADIUS), 0),
                                    _gk(_SIGMA, _RADIUS), 1), _gk(_SIGMA, _RADIUS), 2)
    ltm_terrain_new = ltm_terrain + _XI_H * blurred
    vnorm = jnp.linalg.norm(stm_V, axis=-1)
    scale = jnp.minimum(1.0, _C_V / (vnorm + 1e-6))
    stm_V_norm = stm_V * scale[:, None]
    fatigue_new = _RHO_F * ((1.0 - _LEAK) * fatigue + jnp.sum(omega))
    return (ltm_K_new, ltm_V_new, ltm_e_new, ltm_h_new, ltm_terrain_new,
            stm_V_norm, fatigue_new, new_count)


# speedup vs baseline: 1.3944x; 1.3944x over previous
import functools

import jax
import jax.numpy as jnp
from jax.experimental import pallas as pl

_TOP_M = 1024
_KAPPA = 0.05
_XI_H = 0.005
_RHO_F = 0.2
_C_V = 2.0
_LEAK = 0.01
_SIGMA = 2.0
_RADIUS = 4
_THRESH = 0.5

_M_LTM = 100000
_BLK = 2048
_NB = (_M_LTM + _BLK - 1) // _BLK
_BIG_I = 2**30

_INTERPRET = False


def _gk(sigma, radius):
    x = jnp.arange(-radius, radius + 1, dtype=jnp.float32)
    k = jnp.exp(-0.5 * (x / sigma) ** 2)
    return k / jnp.sum(k)


def _blur_axis(t, kern, axis):
    r = (kern.shape[0] - 1) // 2
    pad = [(0, 0)] * t.ndim
    pad[axis] = (r, r)
    tp = jnp.pad(t, pad)
    L = t.shape[axis]
    out = jnp.zeros_like(t)
    for j in range(kern.shape[0]):
        sl = [slice(None)] * t.ndim
        sl[axis] = slice(j, j + L)
        out = out + kern[j] * tp[tuple(sl)]
    return out


def _bf16_dot(a, b):
    return jax.lax.dot(a.astype(jnp.bfloat16), b.astype(jnp.bfloat16),
                       preferred_element_type=jnp.float32)


def _argmax_body(qn_ref, ltm_ref, val_ref, idx_ref, cnt_ref):
    i = pl.program_id(0)

    @pl.when(i == 0)
    def _init():
        val_ref[...] = jnp.full_like(val_ref, -jnp.inf)
        idx_ref[...] = jnp.zeros_like(idx_ref)

    x = ltm_ref[...]
    norm = jnp.sqrt(jnp.sum(x * x, axis=1, keepdims=True))
    kn = x / (norm + 1e-6)
    sim = jax.lax.dot_general(
        qn_ref[...].astype(jnp.bfloat16), kn.astype(jnp.bfloat16),
        ((((1,), (1,))), ((), ())), preferred_element_type=jnp.float32)
    base = i * _BLK
    col = base + jax.lax.broadcasted_iota(jnp.int32, sim.shape, 1)
    sim = jnp.where(col < _M_LTM, sim, -jnp.inf)
    blk_max = jnp.max(sim, axis=1, keepdims=True)
    blk_idx = jnp.min(jnp.where(sim == blk_max, col, _BIG_I), axis=1,
                      keepdims=True)
    upd = blk_max > val_ref[...]
    val_ref[...] = jnp.where(upd, blk_max, val_ref[...])
    idx_ref[...] = jnp.where(upd, blk_idx, idx_ref[...])

    @pl.when(i == _NB - 1)
    def _fin():
        cnt_ref[...] = jnp.sum(
            (val_ref[...] <= _THRESH).astype(jnp.int32)).reshape(1, 1)


def _sim_argmax(qn, ltm_K):
    val, idx, cnt = pl.pallas_call(
        _argmax_body,
        grid=(_NB,),
        in_specs=[
            pl.BlockSpec((_TOP_M, 64), lambda i: (0, 0)),
            pl.BlockSpec((_BLK, 64), lambda i: (i, 0)),
        ],
        out_specs=[
            pl.BlockSpec((_TOP_M, 1), lambda i: (0, 0)),
            pl.BlockSpec((_TOP_M, 1), lambda i: (0, 0)),
            pl.BlockSpec((1, 1), lambda i: (0, 0)),
        ],
        out_shape=[
            jax.ShapeDtypeStruct((_TOP_M, 1), jnp.float32),
            jax.ShapeDtypeStruct((_TOP_M, 1), jnp.int32),
            jax.ShapeDtypeStruct((1, 1), jnp.int32),
        ],
        interpret=_INTERPRET,
    )(qn, ltm_K)
    return val[:, 0], idx[:, 0], cnt[0, 0]


def kernel(stm_K, stm_V, stm_e, stm_h, stm_active, ltm_K, ltm_V, ltm_e, ltm_h,
           stm_terrain, ltm_terrain, fatigue, W, b):
    masked_h = jnp.where(stm_active, stm_h, -jnp.inf)
    top_h, top_idx = jax.lax.top_k(masked_h, _TOP_M)
    K_sel = stm_K[top_idx]
    V_sel = stm_V[top_idx]
    e_sel = stm_e[top_idx]
    h_sel = stm_h[top_idx]
    K_proj = _bf16_dot(K_sel, W) + b
    omega = _KAPPA * h_sel
    qn = K_proj / (jnp.linalg.norm(K_proj, axis=-1, keepdims=True) + 1e-6)
    best_sim, best_idx, new_count = _sim_argmax(qn, ltm_K)
    alpha = omega[:, None]
    ltm_V_new = ltm_V.at[best_idx].add(alpha * V_sel)
    ltm_K_new = ltm_K.at[best_idx].add(alpha * K_proj)
    ltm_e_new = ltm_e.at[best_idx].add(alpha * e_sel)
    ltm_h_new = ltm_h.at[best_idx].add(omega)
    blurred = _blur_axis(_blur_axis(_blur_axis(stm_terrain, _gk(_SIGMA, _RADIUS), 0),
                                    _gk(_SIGMA, _RADIUS), 1), _gk(_SIGMA, _RADIUS), 2)
    ltm_terrain_new = ltm_terrain + _XI_H * blurred
    vnorm = jnp.linalg.norm(stm_V, axis=-1)
    scale = jnp.minimum(1.0, _C_V / (vnorm + 1e-6))
    stm_V_norm = stm_V * scale[:, None]
    fatigue_new = _RHO_F * ((1.0 - _LEAK) * fatigue + jnp.sum(omega))
    return (ltm_K_new, ltm_V_new, ltm_e_new, ltm_h_new, ltm_terrain_new,
            stm_V_norm, fatigue_new, new_count)
